# Initial kernel scaffold; baseline (speedup 1.0000x reference)
import jax
import jax.numpy as jnp
from jax.experimental import pallas as pl


def kernel(inputs, table0, table1, table2, table3):
    raise NotImplementedError("write your pallas kernel here")


# speedup vs baseline: 5.4548x; 5.4548x over previous
import functools

import jax
import jax.numpy as jnp
from jax import lax
from jax.experimental import pallas as pl
from jax.experimental.pallas import tpu as pltpu
from jax.experimental.pallas import tpu_sc as plsc

NUM_TABLES = 4
VOCAB = 100000
EMB = 32
B = 4096
L = 50

ROWS = B * L
IDX_BLK = 128
PAD = 128


def _sc_gather_concat(i0, i1, i2, i3, t0, t1, t2, t3):
    info = plsc.get_sparse_core_info()
    nc, ns = info.num_cores, info.num_subcores
    nw = nc * ns
    rows_per_w = ROWS // nw
    n_blk = rows_per_w // IDX_BLK

    mesh = plsc.VectorSubcoreMesh(core_axis_name="c", subcore_axis_name="s")

    @functools.partial(
        pl.kernel,
        mesh=mesh,
        compiler_params=pltpu.CompilerParams(use_tc_tiling_on_sc=False),
        out_type=jax.ShapeDtypeStruct((ROWS, NUM_TABLES * EMB), jnp.float32),
        scratch_types=[
            pltpu.VMEM((NUM_TABLES, rows_per_w), jnp.int32),
            pltpu.VMEM((NUM_TABLES, IDX_BLK, EMB), jnp.float32),
            pltpu.VMEM((IDX_BLK, NUM_TABLES * EMB), jnp.float32),
            pltpu.SemaphoreType.DMA,
        ],
    )
    def k(x0, x1, x2, x3, tb0, tb1, tb2, tb3, out_hbm, idx_v, tmp_v, blk_v,
          sem):
        idxs = (x0, x1, x2, x3)
        tables = (tb0, tb1, tb2, tb3)
        wid = lax.axis_index("s") * nc + lax.axis_index("c")
        base = wid * rows_per_w

        for t in range(NUM_TABLES):
            pltpu.sync_copy(idxs[t].at[pl.ds(base, rows_per_w)], idx_v.at[t])

        def body(blk, _):
            descs = []
            for t in range(NUM_TABLES):
                descs.append(pltpu.async_copy(
                    tables[t].at[idx_v.at[t, pl.ds(blk * IDX_BLK, IDX_BLK)]],
                    tmp_v.at[t],
                    sem))
            for d in descs:
                d.wait()
            def crow(r, _):
                for t in range(NUM_TABLES):
                    for h in range(EMB // 16):
                        blk_v[r, pl.ds(t * EMB + h * 16, 16)] = (
                            tmp_v[t, r, pl.ds(h * 16, 16)])
                return ()

            lax.fori_loop(0, IDX_BLK, crow, (), unroll=4)
            pltpu.sync_copy(blk_v, out_hbm.at[pl.ds(base + blk * IDX_BLK,
                                                    IDX_BLK)])
            return ()

        lax.fori_loop(0, n_blk, body, (), unroll=False)

    return k(i0, i1, i2, i3, t0, t1, t2, t3)


def kernel(inputs, table0, table1, table2, table3):
    idx = inputs.astype(jnp.int32).reshape(NUM_TABLES, ROWS)
    out = _sc_gather_concat(idx[0], idx[1], idx[2], idx[3],
                            table0, table1, table2, table3)
    return out.reshape(B, L, NUM_TABLES * EMB)


# speedup vs baseline: 6.3915x; 1.1717x over previous
import functools

import jax
import jax.numpy as jnp
from jax import lax
from jax.experimental import pallas as pl
from jax.experimental.pallas import tpu as pltpu
from jax.experimental.pallas import tpu_sc as plsc

NUM_TABLES = 4
VOCAB = 100000
EMB = 32
B = 4096
L = 50

ROWS = B * L
IDX_BLK = 128


def _sc_gather_concat(i0, i1, i2, i3, t0, t1, t2, t3):
    info = plsc.get_sparse_core_info()
    nc, ns = info.num_cores, info.num_subcores
    nw = nc * ns
    rows_per_w = ROWS // nw
    n_blk = rows_per_w // IDX_BLK

    mesh = plsc.VectorSubcoreMesh(core_axis_name="c", subcore_axis_name="s")

    @functools.partial(
        pl.kernel,
        mesh=mesh,
        compiler_params=pltpu.CompilerParams(use_tc_tiling_on_sc=False),
        out_type=jax.ShapeDtypeStruct((ROWS, NUM_TABLES * EMB), jnp.float32),
        scratch_types=[
            pltpu.VMEM((NUM_TABLES, rows_per_w), jnp.int32),
            pltpu.VMEM((2, NUM_TABLES, IDX_BLK, EMB), jnp.float32),
            pltpu.VMEM((2, IDX_BLK, NUM_TABLES * EMB), jnp.float32),
            pltpu.SemaphoreType.DMA((2,)),
            pltpu.SemaphoreType.DMA((2,)),
        ],
    )
    def k(x0, x1, x2, x3, tb0, tb1, tb2, tb3, out_hbm, idx_v, tmp_v, blk_v,
          gsem, wsem):
        idxs = (x0, x1, x2, x3)
        tables = (tb0, tb1, tb2, tb3)
        wid = lax.axis_index("s") * nc + lax.axis_index("c")
        base = wid * rows_per_w

        for t in range(NUM_TABLES):
            pltpu.sync_copy(idxs[t].at[pl.ds(base, rows_per_w)], idx_v.at[t])

        def gdescs(blk, p):
            return [pltpu.make_async_copy(
                tables[t].at[idx_v.at[t, pl.ds(blk * IDX_BLK, IDX_BLK)]],
                tmp_v.at[p, t],
                gsem.at[p]) for t in range(NUM_TABLES)]

        def wdesc(blk, p):
            return pltpu.make_async_copy(
                blk_v.at[p],
                out_hbm.at[pl.ds(base + blk * IDX_BLK, IDX_BLK)],
                wsem.at[p])

        for d in gdescs(0, 0):
            d.start()

        def body(blk, _):
            p = lax.rem(blk, 2)

            @pl.when(blk + 1 < n_blk)
            def _():
                for d in gdescs(blk + 1, 1 - p):
                    d.start()

            for d in gdescs(blk, p):
                d.wait()

            @pl.when(blk >= 2)
            def _():
                wdesc(blk - 2, p).wait()

            def crow(r, _):
                for t in range(NUM_TABLES):
                    for h in range(EMB // 16):
                        blk_v[p, r, pl.ds(t * EMB + h * 16, 16)] = (
                            tmp_v[p, t, r, pl.ds(h * 16, 16)])
                return ()

            lax.fori_loop(0, IDX_BLK, crow, (), unroll=4)
            wdesc(blk, p).start()
            return ()

        lax.fori_loop(0, n_blk, body, (), unroll=False)
        wdesc(n_blk - 2, 0).wait()
        wdesc(n_blk - 1, 1).wait()

    return k(i0, i1, i2, i3, t0, t1, t2, t3)


def kernel(inputs, table0, table1, table2, table3):
    idx = inputs.astype(jnp.int32).reshape(NUM_TABLES, ROWS)
    out = _sc_gather_concat(idx[0], idx[1], idx[2], idx[3],
                            table0, table1, table2, table3)
    return out.reshape(B, L, NUM_TABLES * EMB)


# speedup vs baseline: 10.1550x; 1.5888x over previous
import functools

import jax
import jax.numpy as jnp
from jax import lax
from jax.experimental import pallas as pl
from jax.experimental.pallas import tpu as pltpu
from jax.experimental.pallas import tpu_sc as plsc

NUM_TABLES = 4
VOCAB = 100000
EMB = 32
B = 4096
L = 50

BLK = 128


def _sc_gather_concat(idxt, t0, t1, t2, t3):
    info = plsc.get_sparse_core_info()
    nc, ns = info.num_cores, info.num_subcores
    nw = nc * ns
    assert B == nw * BLK

    mesh = plsc.VectorSubcoreMesh(core_axis_name="c", subcore_axis_name="s")

    @functools.partial(
        pl.kernel,
        mesh=mesh,
        compiler_params=pltpu.CompilerParams(use_tc_tiling_on_sc=False),
        out_type=jax.ShapeDtypeStruct((L, B, NUM_TABLES * EMB), jnp.float32),
        scratch_types=[
            pltpu.VMEM((L, NUM_TABLES, BLK), jnp.int32),
            pltpu.VMEM((2, NUM_TABLES, BLK, EMB), jnp.float32),
            pltpu.VMEM((2, BLK, NUM_TABLES * EMB), jnp.float32),
            pltpu.SemaphoreType.DMA((2,)),
            pltpu.SemaphoreType.DMA((2,)),
        ],
    )
    def k(idx_hbm, tb0, tb1, tb2, tb3, out_hbm, idx_v, tmp_v, blk_v,
          gsem, wsem):
        tables = (tb0, tb1, tb2, tb3)
        wid = lax.axis_index("s") * nc + lax.axis_index("c")
        b0 = wid * BLK

        pltpu.sync_copy(idx_hbm.at[:, :, pl.ds(b0, BLK)], idx_v)

        def gdescs(l, p):
            return [pltpu.make_async_copy(
                tables[t].at[idx_v.at[l, t]],
                tmp_v.at[p, t],
                gsem.at[p]) for t in range(NUM_TABLES)]

        def wdesc(l, p):
            return pltpu.make_async_copy(
                blk_v.at[p],
                out_hbm.at[l, pl.ds(b0, BLK)],
                wsem.at[p])

        for d in gdescs(0, 0):
            d.start()

        def body(l, _):
            p = lax.rem(l, 2)

            @pl.when(l + 1 < L)
            def _():
                for d in gdescs(l + 1, 1 - p):
                    d.start()

            for d in gdescs(l, p):
                d.wait()

            @pl.when(l >= 2)
            def _():
                wdesc(l - 2, p).wait()

            def crow(r, _):
                for t in range(NUM_TABLES):
                    for h in range(EMB // 16):
                        blk_v[p, r, pl.ds(t * EMB + h * 16, 16)] = (
                            tmp_v[p, t, r, pl.ds(h * 16, 16)])
                return ()

            lax.fori_loop(0, BLK, crow, (), unroll=4)
            wdesc(l, p).start()
            return ()

        lax.fori_loop(0, L, body, (), unroll=False)
        wdesc(L - 2, 0).wait()
        wdesc(L - 1, 1).wait()

    return k(idxt, t0, t1, t2, t3)


def kernel(inputs, table0, table1, table2, table3):
    idxt = inputs.astype(jnp.int32).transpose(2, 0, 1)
    out = _sc_gather_concat(idxt, table0, table1, table2, table3)
    return out.transpose(1, 0, 2)
